# baseline (speedup 1.0000x reference)
import jax
import jax.numpy as jnp
from jax.experimental import pallas as pl
from jax.experimental.pallas import tpu as pltpu

N = 2048
F_IN = 512
F_OUT = 8
CHUNK_ROWS = 128
NCHUNK = N // CHUNK_ROWS
ALPHA = 0.2


def _adj_copy(adj_hbm, adj_buf, sem, k):
    return pltpu.make_async_copy(
        adj_hbm.at[pl.ds(k * CHUNK_ROWS, CHUNK_ROWS), :],
        adj_buf.at[pl.ds(k * CHUNK_ROWS, CHUNK_ROWS), :],
        sem.at[k],
    )


def _gat_kernel(x_ref, adj_hbm, w_ref, a_ref, out_ref, adj_buf, h9_ref, bd_ref, ac_ref, sem):
    for k in range(NCHUNK):
        _adj_copy(adj_hbm, adj_buf, sem, k).start()

    h = jnp.dot(x_ref[...], w_ref[...], preferred_element_type=jnp.float32)
    ones = jnp.ones((N, 1), dtype=jnp.float32)
    zeros = jnp.zeros((N, 7), dtype=jnp.float32)
    h9_ref[...] = jnp.concatenate([h, ones, zeros], axis=1)
    a_src = a_ref[0, :F_OUT].reshape(F_OUT, 1)
    a_dst = a_ref[0, F_OUT:].reshape(F_OUT, 1)
    s = jnp.dot(h, a_src, preferred_element_type=jnp.float32)
    d = jnp.dot(h, a_dst, preferred_element_type=jnp.float32)
    ac_ref[...] = jnp.concatenate([jnp.exp(-s), jnp.exp(-ALPHA * s)], axis=1)
    d_row = d.reshape(1, N)
    bd_ref[...] = jnp.concatenate([jnp.exp(-d_row), jnp.exp(-ALPHA * d_row)], axis=0)

    B = bd_ref[0:1, :]
    D = bd_ref[1:2, :]
    h9 = h9_ref[...]
    for k in range(NCHUNK):
        _adj_copy(adj_hbm, adj_buf, sem, k).wait()
        rows = pl.ds(k * CHUNK_ROWS, CHUNK_ROWS)
        A = ac_ref[rows, 0:1]
        C = ac_ref[rows, 1:2]
        mask = adj_buf[rows, :].astype(jnp.float32)
        e = mask * jnp.minimum(A * B, C * D)
        agg = jnp.dot(e, h9, preferred_element_type=jnp.float32)
        v = agg[:, :F_OUT] / agg[:, F_OUT : F_OUT + 1]
        out_ref[rows, :] = jnp.where(v > 0, v, jnp.exp(jnp.minimum(v, 0.0)) - 1.0)


@jax.jit
def kernel(input, adj, W, a):
    return pl.pallas_call(
        _gat_kernel,
        in_specs=[
            pl.BlockSpec((N, F_IN), lambda: (0, 0)),
            pl.BlockSpec(memory_space=pltpu.MemorySpace.HBM),
            pl.BlockSpec((F_IN, F_OUT), lambda: (0, 0)),
            pl.BlockSpec((1, 2 * F_OUT), lambda: (0, 0)),
        ],
        out_specs=pl.BlockSpec((N, F_OUT), lambda: (0, 0)),
        out_shape=jax.ShapeDtypeStruct((N, F_OUT), jnp.float32),
        scratch_shapes=[
            pltpu.VMEM((N, N), jnp.int32),
            pltpu.VMEM((N, 2 * F_OUT), jnp.float32),
            pltpu.VMEM((2, N), jnp.float32),
            pltpu.VMEM((N, 2), jnp.float32),
            pltpu.SemaphoreType.DMA((NCHUNK,)),
        ],
    )(input, adj, W, a)
